# baseline (speedup 1.0000x reference)
import functools

import jax
import jax.numpy as jnp
from jax import lax
from jax.experimental import pallas as pl
from jax.experimental.pallas import tpu as pltpu
from jax.experimental.pallas import tpu_sc as plsc

_ROWS = 50000
_COLS = 256
_NSEL = 16
_L = 16
_NC = 2
_NS = 16
_NW = _NC * _NS
_CH = 200
_NCHUNK = _ROWS // _CH
_NITER = -(-_NCHUNK // _NW)
_SUB = ((0, 48), (48, 48), (96, 48), (144, 56))
_NSUB = len(_SUB)


def _sc_body(x_hbm, mean_hbm, scale_hbm, out_hbm,
             buf0, buf1, mean_v, scale_v, *sems):
    isems = (sems[0:_NSUB], sems[_NSUB:2 * _NSUB])
    osems = (sems[2 * _NSUB], sems[2 * _NSUB + 1])
    bufs = (buf0, buf1)

    wid = lax.axis_index("s") * _NC + lax.axis_index("c")

    def row0(i):
        return (wid + i * _NW) * _CH

    def start_in(i, s):
        r = row0(i)
        for q, (o, n) in enumerate(_SUB):
            pltpu.async_copy(x_hbm.at[pl.ds(r + o, n)],
                             bufs[s].at[pl.ds(o, n)], isems[s][q])

    def wait_in_q(s, q):
        o, n = _SUB[q]
        pltpu.make_async_copy(x_hbm.at[pl.ds(0, n)],
                              bufs[s].at[pl.ds(o, n)], isems[s][q]).wait()

    def wait_out(s):
        pltpu.make_async_copy(bufs[s], out_hbm.at[pl.ds(0, _CH)],
                              osems[s]).wait()

    start_in(0, 0)

    pltpu.sync_copy(mean_hbm, mean_v)
    pltpu.sync_copy(scale_hbm, scale_v)
    mv = mean_v[...]
    rsv = 1.0 / scale_v[...]

    lane = lax.broadcasted_iota(jnp.int32, (_L,), 0)
    perm = (_L - lane) & (_L - 1)
    is0 = lane == 0
    _dnums = lax.GatherDimensionNumbers(
        offset_dims=(), collapsed_slice_dims=(0,), start_index_map=(0,))

    def _permute(v):
        return lax.gather(v, perm[:, None], _dnums, slice_sizes=(1,),
                          mode=lax.GatherScatterMode.PROMISE_IN_BOUNDS)

    def compute_q(buf, q):
        o, n = _SUB[q]

        def fix_row(r, carry):
            head = buf[r, pl.ds(0, _L)]
            tail = buf[r, pl.ds(_COLS - _L, _L)]
            g = _permute(tail)
            src = jnp.where(is0, head, g)
            buf[r, pl.ds(0, _L)] = (src - mv) * rsv
            return carry

        lax.fori_loop(o, o + n, fix_row, 0)

    def process(i, s):
        r = row0(i)
        for q, (o, n) in enumerate(_SUB):
            wait_in_q(s, q)
            compute_q(bufs[s], q)
            pltpu.async_copy(bufs[s].at[pl.ds(o, n)],
                             out_hbm.at[pl.ds(r + o, n)], osems[s])

    last = _NITER - 1
    has_last = wid < _NCHUNK - last * _NW

    for i in range(last):
        s = i & 1
        if i >= 1:
            wait_out(1 - s)
        if i + 1 < last:
            start_in(i + 1, 1 - s)
        elif i + 1 == last:
            @pl.when(has_last)
            def _():
                start_in(last, 1 - s)
        process(i, s)
    wait_out((last - 1) & 1)

    @pl.when(has_last)
    def _():
        s = last & 1
        process(last, s)
        wait_out(s)


@functools.partial(jax.jit, static_argnames=())
def _sc_transform(X, mean, scale):
    mesh = plsc.VectorSubcoreMesh(core_axis_name="c", subcore_axis_name="s")
    return pl.kernel(
        _sc_body,
        out_type=jax.ShapeDtypeStruct((_ROWS, _COLS), jnp.float32),
        mesh=mesh,
        scratch_types=[
            pltpu.VMEM((_CH, _COLS), jnp.float32),
            pltpu.VMEM((_CH, _COLS), jnp.float32),
            pltpu.VMEM((_L,), jnp.float32),
            pltpu.VMEM((_L,), jnp.float32),
        ] + [pltpu.SemaphoreType.DMA] * (2 * _NSUB + 2),
    )(X, mean, scale)


def kernel(X, mean, scale, inds):
    del inds
    return _sc_transform(X, mean, scale)


# speedup vs baseline: 1.0301x; 1.0301x over previous
import functools

import jax
import jax.numpy as jnp
from jax import lax
from jax.experimental import pallas as pl
from jax.experimental.pallas import tpu as pltpu
from jax.experimental.pallas import tpu_sc as plsc

_ROWS = 50000
_COLS = 256
_NSEL = 16
_L = 16
_NC = 2
_NS = 16
_NW = _NC * _NS
_CH = 200
_NCHUNK = _ROWS // _CH
_NITER = -(-_NCHUNK // _NW)


_SUB = ((0, 48), (48, 48), (96, 48), (144, 56))


def _sc_body(x_hbm, mean_hbm, scale_hbm, out_hbm,
             buf0, buf1, mean_v, scale_v, isem0, isem1, osem0, osem1,
             qsem0, qsem1, qsem2, qsem3):
    wid = lax.axis_index("s") * _NC + lax.axis_index("c")
    qsems = (qsem0, qsem1, qsem2, qsem3)

    def start_in_quartered(r, buf):
        for q, (o, n) in enumerate(_SUB):
            pltpu.async_copy(x_hbm.at[pl.ds(r + o, n)],
                             buf.at[pl.ds(o, n)], qsems[q])

    def wait_in_quarter(buf, q):
        o, n = _SUB[q]
        pltpu.make_async_copy(x_hbm.at[pl.ds(0, n)],
                              buf.at[pl.ds(o, n)], qsems[q]).wait()

    start_in_quartered(wid * _CH, buf0)

    pltpu.sync_copy(mean_hbm, mean_v)
    pltpu.sync_copy(scale_hbm, scale_v)
    mv = mean_v[...]
    rsv = 1.0 / scale_v[...]

    lane = lax.broadcasted_iota(jnp.int32, (_L,), 0)
    perm = (_L - lane) & (_L - 1)
    is0 = lane == 0
    _dnums = lax.GatherDimensionNumbers(
        offset_dims=(), collapsed_slice_dims=(0,), start_index_map=(0,))

    def _permute(v):
        return lax.gather(v, perm[:, None], _dnums, slice_sizes=(1,),
                          mode=lax.GatherScatterMode.PROMISE_IN_BOUNDS)

    bufs = (buf0, buf1)
    isems = (isem0, isem1)
    osems = (osem0, osem1)

    def row0(i):
        return (wid + i * _NW) * _CH

    def compute_rows(buf, lo, hi):
        def fix_row(r, carry):
            head = buf[r, pl.ds(0, _L)]
            tail = buf[r, pl.ds(_COLS - _L, _L)]
            g = _permute(tail)
            src = jnp.where(is0, head, g)
            buf[r, pl.ds(0, _L)] = (src - mv) * rsv
            return carry

        lax.fori_loop(lo, hi, fix_row, 0)

    def compute(buf):
        compute_rows(buf, 0, _CH)

    last = _NITER - 1
    has_last = wid < _NCHUNK - last * _NW

    in_d = [None] * _NITER
    out_d = [None] * _NITER
    for i in range(last):
        s = i & 1
        if i >= 1:
            out_d[i - 1].wait()
        if i + 1 < last:
            in_d[i + 1] = pltpu.async_copy(
                x_hbm.at[pl.ds(row0(i + 1), _CH)], bufs[1 - s], isems[1 - s])
        elif i + 1 == last:
            @pl.when(has_last)
            def _():
                start_in_quartered(row0(last), bufs[1 - s])
        if i == 0:
            for q, (o, n) in enumerate(_SUB):
                wait_in_quarter(bufs[0], q)
                compute_rows(bufs[0], o, o + n)
        else:
            in_d[i].wait()
            compute(bufs[s])
        out_d[i] = pltpu.async_copy(
            bufs[s], out_hbm.at[pl.ds(row0(i), _CH)], osems[s])
    out_d[last - 1].wait()

    @pl.when(has_last)
    def _():
        s = last & 1
        r = row0(last)
        for q, (o, n) in enumerate(_SUB):
            wait_in_quarter(bufs[s], q)
            compute_rows(bufs[s], o, o + n)
            pltpu.async_copy(bufs[s].at[pl.ds(o, n)],
                             out_hbm.at[pl.ds(r + o, n)], osems[s])
        pltpu.make_async_copy(
            bufs[s], out_hbm.at[pl.ds(0, _CH)], osems[s]).wait()


@functools.partial(jax.jit, static_argnames=())
def _sc_transform(X, mean, scale):
    mesh = plsc.VectorSubcoreMesh(core_axis_name="c", subcore_axis_name="s")
    return pl.kernel(
        _sc_body,
        out_type=jax.ShapeDtypeStruct((_ROWS, _COLS), jnp.float32),
        mesh=mesh,
        scratch_types=[
            pltpu.VMEM((_CH, _COLS), jnp.float32),
            pltpu.VMEM((_CH, _COLS), jnp.float32),
            pltpu.VMEM((_L,), jnp.float32),
            pltpu.VMEM((_L,), jnp.float32),
        ] + [pltpu.SemaphoreType.DMA] * 8,
    )(X, mean, scale)


def kernel(X, mean, scale, inds):
    del inds
    return _sc_transform(X, mean, scale)
